# Initial kernel scaffold; baseline (speedup 1.0000x reference)
import jax
import jax.numpy as jnp
from jax.experimental import pallas as pl


def kernel(x, w1, b1, w2, b2):
    raise NotImplementedError("write your pallas kernel here")


# speedup vs baseline: 9.5368x; 9.5368x over previous
import numpy as np
import jax
import jax.numpy as jnp
from jax import lax
from jax.experimental import pallas as pl
from jax.experimental.pallas import tpu as pltpu

_B = 8


def _subpix_kernel(x_ref, w_ref, b_ref, o_ref, xcol_ref):
    B, C, HW = x_ref.shape
    x2 = x_ref[...].reshape(B * C, HW)

    idx = lax.broadcasted_iota(jnp.int32, (1, HW), 1)
    ii = idx // 16
    jj = idx % 16

    for dh in (-1, 0, 1):
        for dw in (-1, 0, 1):
            t = (dh + 1) * 3 + (dw + 1)
            s = dh * 16 + dw
            shifted = jnp.roll(x2, -s, axis=1) if s % HW else x2
            valid = ((ii + dh >= 0) & (ii + dh < 16)
                     & (jj + dw >= 0) & (jj + dw < 16))
            masked = jnp.where(valid, shifted, jnp.bfloat16(0))
            for b in range(B):
                xcol_ref[t * C:(t + 1) * C, b * HW:(b + 1) * HW] = (
                    masked[b * C:(b + 1) * C, :])

    acc = jnp.dot(w_ref[...], xcol_ref[...],
                  preferred_element_type=jnp.float32)
    acc = acc + b_ref[...]
    for b in range(B):
        o_ref[b] = acc[:, b * HW:(b + 1) * HW]


def _pack_weights(w1, b1, w2, b2):
    C = w1.shape[0]
    w1 = jnp.asarray(w1, jnp.float32)
    w2 = jnp.asarray(w2, jnp.float32)
    w5 = w2 + jnp.pad(w1, ((0, 0), (0, 0), (1, 1), (1, 1)))
    groups = {0: ((0, 1), (2, 3), (4,)), 1: ((0,), (1, 2), (3, 4))}
    w_eff = jnp.zeros((2, 2, C, C, 3, 3), jnp.float32)
    for a in (0, 1):
        for bb in (0, 1):
            for di, khs in enumerate(groups[a]):
                for ei, kws in enumerate(groups[bb]):
                    tap = sum(w5[:, :, kh, kw] for kh in khs for kw in kws)
                    w_eff = w_eff.at[a, bb, :, :, di, ei].set(tap)
    w_all = jnp.transpose(w_eff, (0, 1, 2, 4, 5, 3)).reshape(4 * C, 9 * C)
    bsum = (jnp.asarray(b1, jnp.float32) + jnp.asarray(b2, jnp.float32))
    b_all = jnp.tile(bsum, (4,)).reshape(4 * C, 1)
    return w_all.astype(jnp.bfloat16), b_all


def kernel(x, w1, b1, w2, b2):
    N, C, H_in, W_in = x.shape
    HW = H_in * W_in
    B = _B
    w_all, b_all = _pack_weights(w1, b1, w2, b2)
    x_flat = jnp.asarray(x, jnp.bfloat16).reshape(N, C, HW)

    out = pl.pallas_call(
        _subpix_kernel,
        out_shape=jax.ShapeDtypeStruct((N, 4 * C, HW), jnp.float32),
        grid=(N // B,),
        in_specs=[
            pl.BlockSpec((B, C, HW), lambda g: (g, 0, 0)),
            pl.BlockSpec((4 * C, 9 * C), lambda g: (0, 0)),
            pl.BlockSpec((4 * C, 1), lambda g: (0, 0)),
        ],
        out_specs=pl.BlockSpec((B, 4 * C, HW), lambda g: (g, 0, 0)),
        scratch_shapes=[pltpu.VMEM((9 * C, B * HW), jnp.bfloat16)],
        compiler_params=pltpu.CompilerParams(
            dimension_semantics=("parallel",)),
    )(x_flat, w_all, b_all)

    out = out.reshape(N, 2, 2, C, H_in, W_in)
    out = jnp.transpose(out, (0, 3, 4, 1, 5, 2))
    return out.reshape(N, C, 2 * H_in, 2 * W_in)
